# baseline (speedup 1.0000x reference)
import functools

import jax
import jax.numpy as jnp
from jax import lax
from jax.experimental import pallas as pl
from jax.experimental.pallas import tpu as pltpu
from jax.experimental.pallas import tpu_sc as plsc

_K = 20
_KP = 32
_T = 512
_P = 16
_NW = 32


def _knn_body(xt_full_ref, xt_tile_ref, wn_ref, wc_ref, idx_ref, y_ref, z_ref,
              acc_ref):
    t = pl.program_id(0)
    xf = xt_full_ref[...]
    xtile = xt_tile_ref[...]
    nn = xf.shape[0]
    tt = xtile.shape[0]
    hi = lax.Precision.HIGHEST
    dn = (((1,), (0,)), ((), ()))
    y = lax.dot_general(xtile, wn_ref[...], dn, precision=hi)
    y_ref[...] = jnp.concatenate([y, y * y], axis=1)
    z_ref[...] = lax.dot_general(xtile, wc_ref[...], dn, precision=hi)

    sqf = jnp.sum(xf * xf, axis=1)
    sqt = jnp.sum(xtile * xtile, axis=1, keepdims=True)
    prod = lax.dot_general(xtile, xf, (((1,), (1,)), ((), ())))
    dist = sqt + sqf[None, :] - 2.0 * prod

    row_ids = t * tt + lax.broadcasted_iota(jnp.int32, (tt, nn), 0)
    col_ids = lax.broadcasted_iota(jnp.int32, (tt, nn), 1)
    inf = jnp.float32(jnp.inf)
    dist = jnp.where(col_ids == row_ids, inf, dist)
    for j in range(_K):
        am = jnp.argmin(dist, axis=1, keepdims=True).astype(jnp.int32)
        acc_ref[:, j:j + 1] = am
        dist = jnp.where(col_ids == am, inf, dist)
    idx_ref[...] = acc_ref[...].T


def _stats_body(s12_ref, z_ref, gamma_ref, beta_ref, scale_ref, shift_ref):
    o = z_ref.shape[1]
    s1 = s12_ref[:, :o]
    s2 = s12_ref[:, o:]
    z = z_ref[...]
    kf = jnp.float32(_K)
    cnt = jnp.float32(z.shape[0] * _K)
    sum1 = jnp.sum(s1 + kf * z, axis=0)
    sum2 = jnp.sum(s2 + 2.0 * z * s1 + kf * z * z, axis=0)
    mean = sum1 / cnt
    var = sum2 / cnt - mean * mean
    scale = gamma_ref[...] / jnp.sqrt(var + 1e-5)
    scale_ref[...] = scale
    shift_ref[...] = beta_ref[...] - mean * scale


def _apply_body(mm_ref, z_ref, scale_ref, shift_ref, out_ref):
    o = z_ref.shape[2]
    mx = mm_ref[0, :, :o]
    mn = mm_ref[0, :, o:]
    z = z_ref[0]
    scale = scale_ref[...]
    shift = shift_ref[...]
    hsel = z + jnp.where(scale >= 0.0, mx, mn)
    res = jnp.maximum(hsel * scale[None, :] + shift[None, :], 0.0)
    out_ref[0] = res.T


def kernel(x, W, gamma, beta, k, knn_chunk_size):
    B, C, N = x.shape
    O = W.shape[0]
    f32 = jnp.float32
    shift = (jnp.asarray(k).astype(f32) - f32(_K))
    xt = jnp.transpose(x, (0, 2, 1)).astype(f32) + shift
    Wn = W[:, :C].T
    Wc = (W[:, C:] - W[:, :C]).T

    nt = N // _T
    knn_call = pl.pallas_call(
        _knn_body,
        grid=(nt,),
        in_specs=[
            pl.BlockSpec((N, C), lambda t: (0, 0)),
            pl.BlockSpec((_T, C), lambda t: (t, 0)),
            pl.BlockSpec((C, O), lambda t: (0, 0)),
            pl.BlockSpec((C, O), lambda t: (0, 0)),
        ],
        out_specs=[
            pl.BlockSpec((_KP, _T), lambda t: (0, t)),
            pl.BlockSpec((_T, 2 * O), lambda t: (t, 0)),
            pl.BlockSpec((_T, O), lambda t: (t, 0)),
        ],
        out_shape=[
            jax.ShapeDtypeStruct((_KP, N), jnp.int32),
            jax.ShapeDtypeStruct((N, 2 * O), f32),
            jax.ShapeDtypeStruct((N, O), f32),
        ],
        scratch_shapes=[pltpu.VMEM((_T, _KP), jnp.int32)],
    )

    rows_per_w = N // _NW
    n_chunks = rows_per_w // _P
    mesh = plsc.VectorSubcoreMesh(core_axis_name="c", subcore_axis_name="s")

    @functools.partial(
        pl.kernel,
        mesh=mesh,
        out_type=[jax.ShapeDtypeStruct((N, 2 * O), f32) for _ in range(2)],
        scratch_types=[
            pltpu.VMEM((_K, rows_per_w), jnp.int32),
            pltpu.VMEM((_K, _P), jnp.int32),
            pltpu.VMEM((_K, _P, 2 * O), f32),
            pltpu.VMEM((_P, 2 * O), f32),
            pltpu.VMEM((_P, 2 * O), f32),
            pltpu.SemaphoreType.DMA,
        ],
    )
    def _sc_gather(yq_hbm, idx_hbm, s12_hbm, mm_hbm,
                   idx_v, idx_c, rows_v, s12_v, mm_v, sem):
        wid = lax.axis_index("c") * 16 + lax.axis_index("s")
        row_base = wid * rows_per_w
        for j in range(_K):
            pltpu.sync_copy(idx_hbm.at[pl.ds(j * N + row_base, rows_per_w)],
                            idx_v.at[j])

        def chunk(ch, carry):
            c0 = ch * _P
            for j in range(_K):
                idx_c[j, pl.ds(0, _P)] = idx_v[j, pl.ds(c0, _P)]
            copies = [
                pltpu.async_copy(yq_hbm.at[idx_c.at[j]], rows_v.at[j], sem)
                for j in range(_K)
            ]
            for cp in copies:
                cp.wait()

            def per_row(p, c2):
                for c in range(O // 16):
                    sl = pl.ds(c * 16, 16)
                    sq = pl.ds(O + c * 16, 16)
                    v = rows_v[0, p, sl]
                    s = v
                    q = v * v
                    hi = v
                    lo = v
                    for j in range(1, _K):
                        v = rows_v[j, p, sl]
                        s = s + v
                        q = q + v * v
                        hi = jnp.maximum(hi, v)
                        lo = jnp.minimum(lo, v)
                    s12_v[p, sl] = s
                    s12_v[p, sq] = q
                    mm_v[p, sl] = hi
                    mm_v[p, sq] = lo
                return c2

            lax.fori_loop(0, _P, per_row, 0)
            row0 = row_base + c0
            pltpu.sync_copy(s12_v, s12_hbm.at[pl.ds(row0, _P)])
            pltpu.sync_copy(mm_v, mm_hbm.at[pl.ds(row0, _P)])
            return carry

        lax.fori_loop(0, n_chunks, chunk, 0)

    s12_l, mm_l = [], []
    z_l = []
    for b in range(B):
        idx_b, yq_b, z_b = knn_call(xt[b], xt[b], Wn, Wc)
        s12_b, mm_b = _sc_gather(yq_b, idx_b.reshape(_KP * N))
        s12_l.append(s12_b)
        mm_l.append(mm_b)
        z_l.append(z_b)

    s12 = jnp.concatenate(s12_l, axis=0)
    mm = jnp.stack(mm_l, axis=0)
    z = jnp.stack(z_l, axis=0)

    scale, shiftv = pl.pallas_call(
        _stats_body,
        out_shape=[jax.ShapeDtypeStruct((O,), f32) for _ in range(2)],
    )(s12, z.reshape(B * N, O), gamma.astype(f32), beta.astype(f32))

    out = pl.pallas_call(
        _apply_body,
        grid=(B,),
        in_specs=[
            pl.BlockSpec((1, N, 2 * O), lambda b: (b, 0, 0)),
            pl.BlockSpec((1, N, O), lambda b: (b, 0, 0)),
            pl.BlockSpec((O,), lambda b: (0,)),
            pl.BlockSpec((O,), lambda b: (0,)),
        ],
        out_specs=pl.BlockSpec((1, O, N), lambda b: (b, 0, 0)),
        out_shape=jax.ShapeDtypeStruct((B, O, N), f32),
    )(mm, z, scale, shiftv)
    return out


# speedup vs baseline: 1.1155x; 1.1155x over previous
import functools

import jax
import jax.numpy as jnp
from jax import lax
from jax.experimental import pallas as pl
from jax.experimental.pallas import tpu as pltpu
from jax.experimental.pallas import tpu_sc as plsc

_K = 20
_KP = 32
_T = 256
_P = 16
_NW = 32


def _knn_body(xt_full_ref, xt_tile_ref, wn_ref, wc_ref, idx_ref, y_ref, z_ref,
              acc_ref):
    t = pl.program_id(0)
    xf = xt_full_ref[...]
    xtile = xt_tile_ref[...]
    nn = xf.shape[0]
    tt = xtile.shape[0]
    hi = lax.Precision.HIGHEST
    dn = (((1,), (0,)), ((), ()))
    y = lax.dot_general(xtile, wn_ref[...], dn, precision=hi)
    y_ref[...] = jnp.concatenate([y, y * y], axis=1)
    z_ref[...] = lax.dot_general(xtile, wc_ref[...], dn, precision=hi)

    sqf = jnp.sum(xf * xf, axis=1)
    sqt = jnp.sum(xtile * xtile, axis=1, keepdims=True)
    prod = lax.dot_general(xtile, xf, (((1,), (1,)), ((), ())))
    dist = sqt + sqf[None, :] - 2.0 * prod

    row_ids = t * tt + lax.broadcasted_iota(jnp.int32, (tt, nn), 0)
    col_ids = lax.broadcasted_iota(jnp.int32, (tt, nn), 1)
    inf = jnp.float32(jnp.inf)
    dist = jnp.where(col_ids == row_ids, inf, dist)
    for j in range(_K):
        am = jnp.argmin(dist, axis=1, keepdims=True).astype(jnp.int32)
        acc_ref[:, j:j + 1] = am
        dist = jnp.where(col_ids == am, inf, dist)
    idx_ref[...] = acc_ref[...].T


def _stats_body(*refs):
    nb = (len(refs) - 4) // 2
    s12_refs = refs[:nb]
    z_refs = refs[nb:2 * nb]
    gamma_ref, beta_ref, scale_ref, shift_ref = refs[2 * nb:]
    o = z_refs[0].shape[1]
    kf = jnp.float32(_K)
    cnt = jnp.float32(z_refs[0].shape[0] * nb * _K)
    sum1 = jnp.float32(0.0)
    sum2 = jnp.float32(0.0)
    for s12_ref, z_ref in zip(s12_refs, z_refs):
        s1 = s12_ref[:, :o]
        s2 = s12_ref[:, o:]
        z = z_ref[...]
        sum1 = sum1 + jnp.sum(s1 + kf * z, axis=0)
        sum2 = sum2 + jnp.sum(s2 + 2.0 * z * s1 + kf * z * z, axis=0)
    mean = sum1 / cnt
    var = sum2 / cnt - mean * mean
    scale = gamma_ref[...] / jnp.sqrt(var + 1e-5)
    scale_ref[...] = scale
    shift_ref[...] = beta_ref[...] - mean * scale


def _apply_body(mm_ref, z_ref, scale_ref, shift_ref, out_ref):
    o = z_ref.shape[2]
    mx = mm_ref[0, :, :o]
    mn = mm_ref[0, :, o:]
    z = z_ref[0]
    scale = scale_ref[...]
    shift = shift_ref[...]
    hsel = z + jnp.where(scale >= 0.0, mx, mn)
    res = jnp.maximum(hsel * scale[None, :] + shift[None, :], 0.0)
    out_ref[0] = res.T


def kernel(x, W, gamma, beta, k, knn_chunk_size):
    B, C, N = x.shape
    O = W.shape[0]
    f32 = jnp.float32
    shift = (jnp.asarray(k).astype(f32) - f32(_K))
    xt = jnp.transpose(x, (0, 2, 1)).astype(f32) + shift
    Wn = W[:, :C].T
    Wc = (W[:, C:] - W[:, :C]).T

    nt = N // _T
    knn_call = pl.pallas_call(
        _knn_body,
        grid=(nt,),
        in_specs=[
            pl.BlockSpec((N, C), lambda t: (0, 0)),
            pl.BlockSpec((_T, C), lambda t: (t, 0)),
            pl.BlockSpec((C, O), lambda t: (0, 0)),
            pl.BlockSpec((C, O), lambda t: (0, 0)),
        ],
        out_specs=[
            pl.BlockSpec((_KP, _T), lambda t: (0, t)),
            pl.BlockSpec((_T, 2 * O), lambda t: (t, 0)),
            pl.BlockSpec((_T, O), lambda t: (t, 0)),
        ],
        out_shape=[
            jax.ShapeDtypeStruct((_KP, N), jnp.int32),
            jax.ShapeDtypeStruct((N, 2 * O), f32),
            jax.ShapeDtypeStruct((N, O), f32),
        ],
        scratch_shapes=[pltpu.VMEM((_T, _KP), jnp.int32)],
    )

    rows_per_w = N // _NW
    n_chunks = rows_per_w // _P
    mesh = plsc.VectorSubcoreMesh(core_axis_name="c", subcore_axis_name="s")

    @functools.partial(
        pl.kernel,
        mesh=mesh,
        out_type=[jax.ShapeDtypeStruct((N, 2 * O), f32) for _ in range(2)],
        scratch_types=[
            pltpu.VMEM((24, N), jnp.int32),
            pltpu.VMEM((_K, _P), jnp.int32),
            pltpu.VMEM((_K, _P, 2 * O), f32),
            pltpu.VMEM((_P, 2 * O), f32),
            pltpu.VMEM((_P, 2 * O), f32),
            pltpu.SemaphoreType.DMA,
        ],
    )
    def _sc_gather(yq_hbm, idx_hbm, s12_hbm, mm_hbm,
                   idx_v, idx_c, rows_v, s12_v, mm_v, sem):
        wid = lax.axis_index("c") * 16 + lax.axis_index("s")
        row_base = wid * rows_per_w
        pltpu.sync_copy(idx_hbm.at[pl.ds(0, 24)], idx_v)

        def chunk(ch, carry):
            c0 = row_base + ch * _P
            for j in range(_K):
                idx_c[j, pl.ds(0, _P)] = idx_v[j, pl.ds(c0, _P)]
            copies = [
                pltpu.async_copy(yq_hbm.at[idx_c.at[j]], rows_v.at[j], sem)
                for j in range(_K)
            ]
            for cp in copies:
                cp.wait()

            def per_row(p, c2):
                for c in range(O // 16):
                    sl = pl.ds(c * 16, 16)
                    sq = pl.ds(O + c * 16, 16)
                    v = rows_v[0, p, sl]
                    s = v
                    q = v * v
                    hi = v
                    lo = v
                    for j in range(1, _K):
                        v = rows_v[j, p, sl]
                        s = s + v
                        q = q + v * v
                        hi = jnp.maximum(hi, v)
                        lo = jnp.minimum(lo, v)
                    s12_v[p, sl] = s
                    s12_v[p, sq] = q
                    mm_v[p, sl] = hi
                    mm_v[p, sq] = lo
                return c2

            lax.fori_loop(0, _P, per_row, 0)
            pltpu.sync_copy(s12_v, s12_hbm.at[pl.ds(c0, _P)])
            pltpu.sync_copy(mm_v, mm_hbm.at[pl.ds(c0, _P)])
            return carry

        lax.fori_loop(0, n_chunks, chunk, 0)

    s12_l, mm_l = [], []
    z_l = []
    for b in range(B):
        idx_b, yq_b, z_b = knn_call(xt[b], xt[b], Wn, Wc)
        s12_b, mm_b = _sc_gather(yq_b, idx_b)
        s12_l.append(s12_b)
        mm_l.append(mm_b)
        z_l.append(z_b)

    mm = jnp.stack(mm_l, axis=0)
    z = jnp.stack(z_l, axis=0)

    scale, shiftv = pl.pallas_call(
        _stats_body,
        out_shape=[jax.ShapeDtypeStruct((O,), f32) for _ in range(2)],
    )(*s12_l, *z_l, gamma.astype(f32), beta.astype(f32))

    out = pl.pallas_call(
        _apply_body,
        grid=(B,),
        in_specs=[
            pl.BlockSpec((1, N, 2 * O), lambda b: (b, 0, 0)),
            pl.BlockSpec((1, N, O), lambda b: (b, 0, 0)),
            pl.BlockSpec((O,), lambda b: (0,)),
            pl.BlockSpec((O,), lambda b: (0,)),
        ],
        out_specs=pl.BlockSpec((1, O, N), lambda b: (b, 0, 0)),
        out_shape=jax.ShapeDtypeStruct((B, O, N), f32),
    )(mm, z, scale, shiftv)
    return out
